# baseline (speedup 1.0000x reference)
import functools

import jax
import jax.numpy as jnp
from jax import lax
from jax.experimental import pallas as pl
from jax.experimental.pallas import tpu as pltpu
from jax.experimental.pallas import tpu_sc as plsc

BATCH = 16384
EMBED = 32
HIDDEN = 64
NROWS = 1000000

_NC = 2
_NS = 16
_NW = _NC * _NS
_CL = 1024
_NCH = 977
_LAST_C = 976
_LAST_W = 512
_TAIL = 999936
_NTAIL = NROWS - _TAIL
_CAND_CAP = 1024
_WL_CAP = 256
_SENT_ROWS = 512
_OUT_ROWS = BATCH + _SENT_ROWS


def _sc_body(uidx_hbm, iidx_hbm, utab_hbm, itab_hbm,
             uout_hbm, iout_hbm,
             idx_v, cand_r, cand_p, wl_r, wl_p, chunk_v, rows16,
             sem_f, sem_s):
    wid = lax.axis_index("s") * _NC + lax.axis_index("c")
    lanes = lax.iota(jnp.int32, 16)
    wid_v = jnp.full((16,), wid, jnp.int32)
    zeros16 = jnp.zeros((16,), jnp.int32)
    tail_v = jnp.full((16,), _TAIL, jnp.int32)

    def one_table(idx_hbm, tab_hbm, out_hbm):
        def fetch(c, q):
            @pl.when(c < _LAST_C)
            def _():
                off = pl.multiple_of(c * _CL, 128)
                pltpu.async_copy(tab_hbm.at[:, pl.ds(off, _CL)],
                                 chunk_v.at[q], sem_f)

            @pl.when(c == _LAST_C)
            def _():
                pltpu.async_copy(
                    tab_hbm.at[:, pl.ds(_LAST_C * _CL, _LAST_W)],
                    chunk_v.at[q, :, pl.ds(0, _LAST_W)], sem_f)

        def fetch_wait(c, q):
            @pl.when(c < _LAST_C)
            def _():
                pltpu.make_async_copy(tab_hbm.at[:, pl.ds(0, _CL)],
                                      chunk_v.at[q], sem_f).wait()

            @pl.when(c == _LAST_C)
            def _():
                pltpu.make_async_copy(
                    tab_hbm.at[:, pl.ds(0, _LAST_W)],
                    chunk_v.at[q, :, pl.ds(0, _LAST_W)], sem_f).wait()

        pltpu.sync_copy(idx_hbm, idx_v)
        fetch(wid, 0)

        def compact_body(g, cnt_v):
            ga = jnp.full((16,), g * 32, jnp.int32) + lanes
            gb = ga + 16
            rva = plsc.load_gather(idx_v, [ga])
            rvb = plsc.load_gather(idx_v, [gb])
            ma = ((rva >> 10) & (_NW - 1) == wid_v) & (rva < tail_v)
            mb = ((rvb >> 10) & (_NW - 1) == wid_v) & (rvb < tail_v)
            pos_a = cnt_v + plsc.cumsum(jnp.where(ma, 1, 0)) - 1
            ma = ma & (pos_a < _CAND_CAP)
            cnt_v = cnt_v + plsc.all_reduce_population_count(ma)
            pos_b = cnt_v + plsc.cumsum(jnp.where(mb, 1, 0)) - 1
            mb = mb & (pos_b < _CAND_CAP)
            plsc.store_scatter(cand_r, [pos_a], rva, mask=ma)
            plsc.store_scatter(cand_p, [pos_a], ga, mask=ma)
            plsc.store_scatter(cand_r, [pos_b], rvb, mask=mb)
            plsc.store_scatter(cand_p, [pos_b], gb, mask=mb)
            return cnt_v + plsc.all_reduce_population_count(mb)

        cnt_v = lax.fori_loop(0, BATCH // 32, compact_body, zeros16,
                              unroll=False)
        n = lax.reduce_max(cnt_v, axes=(0,))
        n_v = jnp.full((16,), n, jnp.int32)
        n_groups = (n + 15) >> 4

        k_hi = jnp.where(wid < _NCH - (_NCH // _NW) * _NW,
                         _NCH // _NW + 1, _NCH // _NW)

        def drain_scatter(i, _):
            pltpu.make_async_copy(out_hbm.at[pl.ds(0, 16)],
                                  rows16.at[0], sem_s).wait()
            return ()

        def chunk_body(k, pend):
            q = k & 1
            c = wid + k * _NW
            cbase_v = jnp.full((16,), c * _CL, jnp.int32)

            @pl.when(k + 1 < k_hi)
            def _():
                fetch(wid + (k + 1) * _NW, (k + 1) & 1)

            fetch_wait(c, q)
            lax.fori_loop(0, pend, drain_scatter, (), unroll=False)

            q_v = jnp.full((16,), q, jnp.int32)
            c_v = jnp.full((16,), c, jnp.int32)

            def scan_body(g, w_v):
                g16 = jnp.full((16,), g * 16, jnp.int32) + lanes
                rv = plsc.load_gather(cand_r, [g16])
                pv = plsc.load_gather(cand_p, [g16])
                valid = g16 < n_v
                m = valid & ((rv >> 10) == c_v)
                mi = jnp.where(m, 1, 0)
                pos = w_v + plsc.cumsum(mi) - 1
                m = m & (pos < _WL_CAP)
                plsc.store_scatter(wl_r, [pos], rv, mask=m)
                plsc.store_scatter(wl_p, [pos], pv, mask=m)
                return w_v + plsc.all_reduce_population_count(m)

            w_v = lax.fori_loop(0, n_groups, scan_body, zeros16,
                                unroll=False)
            wcnt = lax.reduce_max(w_v, axes=(0,))

            def emit_body(h, _):
                hq = h & 1
                hq_v = jnp.full((16,), hq, jnp.int32)

                @pl.when(h >= 2)
                def _():
                    pltpu.make_async_copy(out_hbm.at[pl.ds(0, 16)],
                                          rows16.at[hq], sem_s).wait()

                h16 = jnp.full((16,), h * 16, jnp.int32) + lanes
                wr = plsc.load_gather(wl_r, [h16])
                wp = plsc.load_gather(wl_p, [h16])
                valid = h16 < jnp.full((16,), wcnt, jnp.int32)
                lane_vec = wr - cbase_v
                sent = jnp.full((16,), BATCH, jnp.int32) + wid_v * 16 + lanes
                p16 = jnp.where(valid, wp, sent)
                for j in range(16):
                    vj = (h * 16 + j) < wcnt

                    @pl.when(vj)
                    def _():
                        lane_s = lax.reduce_sum(
                            jnp.where(lanes == j, lane_vec, 0), axes=(0,))
                        lane_sv = jnp.full((16,), lane_s, jnp.int32)
                        j_v = jnp.full((16,), j, jnp.int32)
                        lo = plsc.load_gather(chunk_v, [q_v, lanes, lane_sv])
                        hi = plsc.load_gather(chunk_v,
                                              [q_v, lanes + 16, lane_sv])
                        plsc.store_scatter(rows16, [hq_v, j_v, lanes], lo)
                        plsc.store_scatter(rows16, [hq_v, j_v, lanes + 16],
                                           hi)

                pltpu.async_copy(rows16.at[hq], out_hbm.at[p16], sem_s)
                return ()

            n_emit = (wcnt + 15) >> 4
            lax.fori_loop(0, n_emit, emit_body, (), unroll=False)
            return jnp.minimum(n_emit, 2)

        pend = lax.fori_loop(0, k_hi, chunk_body, 0, unroll=False)
        lax.fori_loop(0, pend, drain_scatter, (), unroll=False)

    one_table(uidx_hbm, utab_hbm, uout_hbm)
    one_table(iidx_hbm, itab_hbm, iout_hbm)


_sc_gather = pl.kernel(
    _sc_body,
    out_type=(
        jax.ShapeDtypeStruct((_OUT_ROWS, 128), jnp.float32),
        jax.ShapeDtypeStruct((_OUT_ROWS, 128), jnp.float32),
    ),
    mesh=plsc.VectorSubcoreMesh(core_axis_name="c", subcore_axis_name="s"),
    scratch_types=[
        pltpu.VMEM((BATCH,), jnp.int32),
        pltpu.VMEM((_CAND_CAP,), jnp.int32),
        pltpu.VMEM((_CAND_CAP,), jnp.int32),
        pltpu.VMEM((_WL_CAP,), jnp.int32),
        pltpu.VMEM((_WL_CAP,), jnp.int32),
        pltpu.VMEM((2, 32, _CL), jnp.float32),
        pltpu.VMEM((2, 16, 128), jnp.float32),
        pltpu.SemaphoreType.DMA,
        pltpu.SemaphoreType.DMA,
    ],
    compiler_params=pltpu.CompilerParams(needs_layout_passes=False),
)


_CHUNK = 4096


def _mlp_body(uidx_ref, vidx_ref, u_ref, v_ref, usub_ref, vsub_ref,
              w1a_ref, w1b_ref, b1_ref, w2_ref, b2_ref, o_ref):
    iu = uidx_ref[...]
    iv = vidx_ref[...]
    tail_ids = lax.broadcasted_iota(jnp.int32, (1, _NTAIL), 1) + _TAIL
    onehot_u = (iu == tail_ids).astype(jnp.float32)
    onehot_v = (iv == tail_ids).astype(jnp.float32)
    u_tail = jnp.dot(onehot_u, usub_ref[...],
                     preferred_element_type=jnp.float32,
                     precision=lax.Precision.HIGHEST)
    v_tail = jnp.dot(onehot_v, vsub_ref[...],
                     preferred_element_type=jnp.float32,
                     precision=lax.Precision.HIGHEST)
    u = jnp.where(iu >= _TAIL, u_tail, u_ref[:, :EMBED])
    v = jnp.where(iv >= _TAIL, v_tail, v_ref[:, :EMBED])
    h = jnp.dot(u, w1a_ref[...],
                preferred_element_type=jnp.float32,
                precision=lax.Precision.HIGHEST)
    h = h + jnp.dot(v, w1b_ref[...],
                    preferred_element_type=jnp.float32,
                    precision=lax.Precision.HIGHEST)
    h = jnp.maximum(h + b1_ref[...], 0.0)
    s = jnp.sum(h * w2_ref[...], axis=1) + b2_ref[0, 0]
    o_ref[...] = 1.0 / (1.0 + jnp.exp(-s))


@functools.partial(jax.jit, donate_argnums=())
def _run(user_indices, item_indices, user_table, item_table, W1, b1, W2, b2):
    upad, ipad = _sc_gather(user_indices, item_indices,
                            user_table.T, item_table.T)

    usub = user_table[_TAIL:]
    vsub = item_table[_TAIL:]
    w1a = W1[:, :EMBED].T
    w1b = W1[:, EMBED:].T
    b1r = b1.reshape(1, HIDDEN)
    b2r = b2.reshape(1, 1)

    grid = BATCH // _CHUNK
    out = pl.pallas_call(
        _mlp_body,
        grid=(grid,),
        in_specs=[
            pl.BlockSpec((_CHUNK, 1), lambda i: (i, 0)),
            pl.BlockSpec((_CHUNK, 1), lambda i: (i, 0)),
            pl.BlockSpec((_CHUNK, 128), lambda i: (i, 0)),
            pl.BlockSpec((_CHUNK, 128), lambda i: (i, 0)),
            pl.BlockSpec((_NTAIL, EMBED), lambda i: (0, 0)),
            pl.BlockSpec((_NTAIL, EMBED), lambda i: (0, 0)),
            pl.BlockSpec((EMBED, HIDDEN), lambda i: (0, 0)),
            pl.BlockSpec((EMBED, HIDDEN), lambda i: (0, 0)),
            pl.BlockSpec((1, HIDDEN), lambda i: (0, 0)),
            pl.BlockSpec((1, HIDDEN), lambda i: (0, 0)),
            pl.BlockSpec((1, 1), lambda i: (0, 0)),
        ],
        out_specs=pl.BlockSpec((_CHUNK,), lambda i: (i,)),
        out_shape=jax.ShapeDtypeStruct((BATCH,), jnp.float32),
    )(user_indices.reshape(BATCH, 1), item_indices.reshape(BATCH, 1),
      upad, ipad, usub, vsub, w1a, w1b, b1r, W2, b2r)
    return out


def kernel(user_indices, item_indices, user_table, item_table, W1, b1, W2, b2):
    return _run(user_indices, item_indices, user_table, item_table,
                W1, b1, W2, b2)


# speedup vs baseline: 1.0187x; 1.0187x over previous
import functools

import jax
import jax.numpy as jnp
from jax import lax
from jax.experimental import pallas as pl
from jax.experimental.pallas import tpu as pltpu
from jax.experimental.pallas import tpu_sc as plsc

BATCH = 16384
EMBED = 32
HIDDEN = 64
NROWS = 1000000

_NC = 2
_NS = 16
_NW = _NC * _NS
_CL = 1024
_NCH = 977
_LAST_C = 976
_LAST_W = 512
_TAIL = 999936
_NTAIL = NROWS - _TAIL
_CAND_CAP = 1024
_WL_CAP = 256
_SENT_ROWS = 512
_OUT_ROWS = BATCH + _SENT_ROWS


def _sc_body(uidx_hbm, iidx_hbm, utab_hbm, itab_hbm,
             uout_hbm, iout_hbm,
             idx_v, cand_r, cand_p, wl_r, wl_p, chunk_v, rows16,
             sem_f, sem_s):
    wid = lax.axis_index("s") * _NC + lax.axis_index("c")
    lanes = lax.iota(jnp.int32, 16)
    wid_v = jnp.full((16,), wid, jnp.int32)
    zeros16 = jnp.zeros((16,), jnp.int32)
    tail_v = jnp.full((16,), _TAIL, jnp.int32)

    def one_table(idx_hbm, tab_hbm, out_hbm):
        def fetch(c, q):
            @pl.when(c < _LAST_C)
            def _():
                off = pl.multiple_of(c * _CL, 128)
                pltpu.async_copy(tab_hbm.at[:, pl.ds(off, _CL)],
                                 chunk_v.at[q], sem_f)

            @pl.when(c == _LAST_C)
            def _():
                pltpu.async_copy(
                    tab_hbm.at[:, pl.ds(_LAST_C * _CL, _LAST_W)],
                    chunk_v.at[q, :, pl.ds(0, _LAST_W)], sem_f)

        def fetch_wait(c, q):
            @pl.when(c < _LAST_C)
            def _():
                pltpu.make_async_copy(tab_hbm.at[:, pl.ds(0, _CL)],
                                      chunk_v.at[q], sem_f).wait()

            @pl.when(c == _LAST_C)
            def _():
                pltpu.make_async_copy(
                    tab_hbm.at[:, pl.ds(0, _LAST_W)],
                    chunk_v.at[q, :, pl.ds(0, _LAST_W)], sem_f).wait()

        pltpu.sync_copy(idx_hbm, idx_v)
        fetch(wid, 0)

        def compact_body(g, cnt_v):
            ga = jnp.full((16,), g * 32, jnp.int32) + lanes
            gb = ga + 16
            rva = plsc.load_gather(idx_v, [ga])
            rvb = plsc.load_gather(idx_v, [gb])
            ma = ((rva >> 10) & (_NW - 1) == wid_v) & (rva < tail_v)
            mb = ((rvb >> 10) & (_NW - 1) == wid_v) & (rvb < tail_v)
            pos_a = cnt_v + plsc.cumsum(jnp.where(ma, 1, 0)) - 1
            ma = ma & (pos_a < _CAND_CAP)
            cnt_v = cnt_v + plsc.all_reduce_population_count(ma)
            pos_b = cnt_v + plsc.cumsum(jnp.where(mb, 1, 0)) - 1
            mb = mb & (pos_b < _CAND_CAP)
            plsc.store_scatter(cand_r, [pos_a], rva, mask=ma)
            plsc.store_scatter(cand_p, [pos_a], ga, mask=ma)
            plsc.store_scatter(cand_r, [pos_b], rvb, mask=mb)
            plsc.store_scatter(cand_p, [pos_b], gb, mask=mb)
            return cnt_v + plsc.all_reduce_population_count(mb)

        cnt_v = lax.fori_loop(0, BATCH // 32, compact_body, zeros16,
                              unroll=False)
        n = lax.reduce_max(cnt_v, axes=(0,))
        n_v = jnp.full((16,), n, jnp.int32)
        n_groups = (n + 15) >> 4

        k_hi = jnp.where(wid < _NCH - (_NCH // _NW) * _NW,
                         _NCH // _NW + 1, _NCH // _NW)

        def drain_scatter(i, _):
            pltpu.make_async_copy(out_hbm.at[pl.ds(0, 16)],
                                  rows16.at[0], sem_s).wait()
            return ()

        def chunk_body(k, pend):
            q = k & 1
            c = wid + k * _NW
            cbase_v = jnp.full((16,), c * _CL, jnp.int32)

            @pl.when(k + 1 < k_hi)
            def _():
                fetch(wid + (k + 1) * _NW, (k + 1) & 1)

            fetch_wait(c, q)
            lax.fori_loop(0, pend, drain_scatter, (), unroll=False)

            q_v = jnp.full((16,), q, jnp.int32)
            c_v = jnp.full((16,), c, jnp.int32)

            def scan_body(g, w_v):
                g16 = jnp.full((16,), g * 16, jnp.int32) + lanes
                rv = plsc.load_gather(cand_r, [g16])
                pv = plsc.load_gather(cand_p, [g16])
                valid = g16 < n_v
                m = valid & ((rv >> 10) == c_v)
                mi = jnp.where(m, 1, 0)
                pos = w_v + plsc.cumsum(mi) - 1
                m = m & (pos < _WL_CAP)
                plsc.store_scatter(wl_r, [pos], rv, mask=m)
                plsc.store_scatter(wl_p, [pos], pv, mask=m)
                return w_v + plsc.all_reduce_population_count(m)

            w_v = lax.fori_loop(0, n_groups, scan_body, zeros16,
                                unroll=False)
            wcnt = lax.reduce_max(w_v, axes=(0,))

            def emit_body(h, _):
                hq = h & 1
                hq_v = jnp.full((16,), hq, jnp.int32)

                @pl.when(h >= 2)
                def _():
                    pltpu.make_async_copy(out_hbm.at[pl.ds(0, 16)],
                                          rows16.at[hq], sem_s).wait()

                h16 = jnp.full((16,), h * 16, jnp.int32) + lanes
                wr = plsc.load_gather(wl_r, [h16])
                wp = plsc.load_gather(wl_p, [h16])
                valid = h16 < jnp.full((16,), wcnt, jnp.int32)
                lane_vec = wr - cbase_v
                sent = jnp.full((16,), BATCH, jnp.int32) + wid_v * 16 + lanes
                p16 = jnp.where(valid, wp, sent)
                for j in range(16):
                    vj = (h * 16 + j) < wcnt

                    @pl.when(vj)
                    def _():
                        lane_s = lax.reduce_sum(
                            jnp.where(lanes == j, lane_vec, 0), axes=(0,))
                        lane_sv = jnp.full((16,), lane_s, jnp.int32)
                        j_v = jnp.full((16,), j, jnp.int32)
                        lo = plsc.load_gather(chunk_v, [q_v, lanes, lane_sv])
                        hi = plsc.load_gather(chunk_v,
                                              [q_v, lanes + 16, lane_sv])
                        plsc.store_scatter(rows16, [hq_v, j_v, lanes], lo)
                        plsc.store_scatter(rows16, [hq_v, j_v, lanes + 16],
                                           hi)

                pltpu.async_copy(rows16.at[hq], out_hbm.at[p16], sem_s)
                return ()

            n_emit = (wcnt + 15) >> 4
            lax.fori_loop(0, n_emit, emit_body, (), unroll=False)
            return jnp.minimum(n_emit, 2)

        pend = lax.fori_loop(0, k_hi, chunk_body, 0, unroll=False)
        lax.fori_loop(0, pend, drain_scatter, (), unroll=False)

    one_table(uidx_hbm, utab_hbm, uout_hbm)
    one_table(iidx_hbm, itab_hbm, iout_hbm)


_sc_gather = pl.kernel(
    _sc_body,
    out_type=(
        jax.ShapeDtypeStruct((_OUT_ROWS, 128), jnp.float32),
        jax.ShapeDtypeStruct((_OUT_ROWS, 128), jnp.float32),
    ),
    mesh=plsc.VectorSubcoreMesh(core_axis_name="c", subcore_axis_name="s"),
    scratch_types=[
        pltpu.VMEM((BATCH,), jnp.int32),
        pltpu.VMEM((_CAND_CAP,), jnp.int32),
        pltpu.VMEM((_CAND_CAP,), jnp.int32),
        pltpu.VMEM((_WL_CAP,), jnp.int32),
        pltpu.VMEM((_WL_CAP,), jnp.int32),
        pltpu.VMEM((2, 32, _CL), jnp.float32),
        pltpu.VMEM((2, 16, 128), jnp.float32),
        pltpu.SemaphoreType.DMA,
        pltpu.SemaphoreType.DMA,
    ],
    compiler_params=pltpu.CompilerParams(needs_layout_passes=False),
)


_CHUNK = 2048


def _mlp_body(uidx_ref, vidx_ref, u_ref, v_ref, usub_ref, vsub_ref,
              w1a_ref, w1b_ref, b1_ref, w2_ref, b2_ref, o_ref):
    iu = uidx_ref[...]
    iv = vidx_ref[...]
    tail_ids = lax.broadcasted_iota(jnp.int32, (1, _NTAIL), 1) + _TAIL
    onehot_u = (iu == tail_ids).astype(jnp.float32)
    onehot_v = (iv == tail_ids).astype(jnp.float32)
    u_tail = jnp.dot(onehot_u, usub_ref[...],
                     preferred_element_type=jnp.float32,
                     precision=lax.Precision.HIGHEST)
    v_tail = jnp.dot(onehot_v, vsub_ref[...],
                     preferred_element_type=jnp.float32,
                     precision=lax.Precision.HIGHEST)
    u = jnp.where(iu >= _TAIL, u_tail, u_ref[:, :EMBED])
    v = jnp.where(iv >= _TAIL, v_tail, v_ref[:, :EMBED])
    h = jnp.dot(u, w1a_ref[...],
                preferred_element_type=jnp.float32,
                precision=lax.Precision.HIGHEST)
    h = h + jnp.dot(v, w1b_ref[...],
                    preferred_element_type=jnp.float32,
                    precision=lax.Precision.HIGHEST)
    h = jnp.maximum(h + b1_ref[...], 0.0)
    s = jnp.sum(h * w2_ref[...], axis=1) + b2_ref[0, 0]
    o_ref[...] = 1.0 / (1.0 + jnp.exp(-s))


@functools.partial(jax.jit, donate_argnums=())
def _run(user_indices, item_indices, user_table, item_table, W1, b1, W2, b2):
    upad, ipad = _sc_gather(user_indices, item_indices,
                            user_table.T, item_table.T)

    usub = user_table[_TAIL:]
    vsub = item_table[_TAIL:]
    w1a = W1[:, :EMBED].T
    w1b = W1[:, EMBED:].T
    b1r = b1.reshape(1, HIDDEN)
    b2r = b2.reshape(1, 1)

    grid = BATCH // _CHUNK
    out = pl.pallas_call(
        _mlp_body,
        grid=(grid,),
        in_specs=[
            pl.BlockSpec((_CHUNK, 1), lambda i: (i, 0)),
            pl.BlockSpec((_CHUNK, 1), lambda i: (i, 0)),
            pl.BlockSpec((_CHUNK, 128), lambda i: (i, 0)),
            pl.BlockSpec((_CHUNK, 128), lambda i: (i, 0)),
            pl.BlockSpec((_NTAIL, EMBED), lambda i: (0, 0)),
            pl.BlockSpec((_NTAIL, EMBED), lambda i: (0, 0)),
            pl.BlockSpec((EMBED, HIDDEN), lambda i: (0, 0)),
            pl.BlockSpec((EMBED, HIDDEN), lambda i: (0, 0)),
            pl.BlockSpec((1, HIDDEN), lambda i: (0, 0)),
            pl.BlockSpec((1, HIDDEN), lambda i: (0, 0)),
            pl.BlockSpec((1, 1), lambda i: (0, 0)),
        ],
        out_specs=pl.BlockSpec((_CHUNK,), lambda i: (i,)),
        out_shape=jax.ShapeDtypeStruct((BATCH,), jnp.float32),
    )(user_indices.reshape(BATCH, 1), item_indices.reshape(BATCH, 1),
      upad, ipad, usub, vsub, w1a, w1b, b1r, W2, b2r)
    return out


def kernel(user_indices, item_indices, user_table, item_table, W1, b1, W2, b2):
    return _run(user_indices, item_indices, user_table, item_table,
                W1, b1, W2, b2)
